# baseline (speedup 1.0000x reference)
import functools

import jax
import jax.numpy as jnp
import numpy as np
from jax import lax
from jax.experimental import pallas as pl
from jax.experimental.pallas import tpu as pltpu
from jax.experimental.pallas import tpu_sc as plsc

LA = 96
LB = 104
BB = 128


def _make_pe_gather(L, D):
    info = plsc.get_sparse_core_info()
    NC = info.num_cores
    mesh = plsc.VectorSubcoreMesh(core_axis_name="c", subcore_axis_name="s")

    @functools.partial(
        pl.kernel,
        mesh=mesh,
        out_type=jax.ShapeDtypeStruct((L, D), jnp.float32),
        scratch_types=[
            pltpu.VMEM((LA,), jnp.int32),
            pltpu.VMEM((LB,), jnp.int32),
            pltpu.VMEM((LA, D), jnp.float32),
            pltpu.VMEM((LB, D), jnp.float32),
            pltpu.SemaphoreType.DMA,
            pltpu.SemaphoreType.DMA,
        ],
    )
    def k(pe_hbm, idx_hbm, out_hbm, idxa_v, idxb_v, bufa, bufb, sema, semb):
        wid = lax.axis_index("s") * NC + lax.axis_index("c")

        @pl.when(wid == 0)
        def _():
            pltpu.sync_copy(idx_hbm.at[pl.ds(0, LA)], idxa_v)
            pltpu.async_copy(pe_hbm.at[idxa_v], bufa, sema).wait()
            pltpu.sync_copy(bufa, out_hbm.at[pl.ds(0, LA)])

        @pl.when(wid == 1)
        def _():
            pltpu.sync_copy(idx_hbm.at[pl.ds(LA, LB)], idxb_v)
            pltpu.async_copy(pe_hbm.at[idxb_v], bufb, semb).wait()
            pltpu.sync_copy(bufb, out_hbm.at[pl.ds(LA, LB)])

    return k


def _tc_add_kernel(x_ref, pe_ref, o_ref):
    o_ref[...] = x_ref[...] + pe_ref[...][None, :, :]


def kernel(x, pe_table):
    B, L, D = x.shape
    position_ids = np.arange(L, dtype=np.int32)
    pe = _make_pe_gather(L, D)(pe_table, position_ids)
    return pl.pallas_call(
        _tc_add_kernel,
        grid=(B // BB,),
        in_specs=[
            pl.BlockSpec((BB, L, D), lambda i: (i, 0, 0)),
            pl.BlockSpec((L, D), lambda i: (0, 0)),
        ],
        out_specs=pl.BlockSpec((BB, L, D), lambda i: (i, 0, 0)),
        out_shape=jax.ShapeDtypeStruct((B, L, D), x.dtype),
    )(x, pe)
